# Initial kernel scaffold; baseline (speedup 1.0000x reference)
import jax
import jax.numpy as jnp
from jax.experimental import pallas as pl


def kernel(x, edge_index, W1, a_src1, a_dst1, b1, W2, a_src2, a_dst2, b2, conv_w, conv_b):
    raise NotImplementedError("write your pallas kernel here")


# speedup vs baseline: 5.7276x; 5.7276x over previous
import functools

import jax
import jax.numpy as jnp
from jax import lax
from jax.experimental import pallas as pl
from jax.experimental.pallas import tpu as pltpu

N = 10000
FM = 128
H = 8
OUT_CH = 128
N_CIRC = 504
N_PAD = 10240
BLK = 256
NEG_SLOPE = 0.2


def _prep_body(W_ref, asrc_ref, adst_ref, AsT_ref, AdT_ref):
    W = W_ref[...]
    for h in range(H):
        Wh = W[:, h * FM:(h + 1) * FM]
        dn = (((1,), (1,)), ((), ()))
        AsT_ref[h:h + 1, :] = lax.dot_general(
            asrc_ref[h:h + 1, :], Wh, dn, preferred_element_type=jnp.float32)
        AdT_ref[h:h + 1, :] = lax.dot_general(
            adst_ref[h:h + 1, :], Wh, dn, preferred_element_type=jnp.float32)


def _prep(W, a_src, a_dst):
    return pl.pallas_call(
        _prep_body,
        out_shape=(jax.ShapeDtypeStruct((H, FM), jnp.float32),
                   jax.ShapeDtypeStruct((H, FM), jnp.float32)),
    )(W, a_src, a_dst)


def _embed1_body(x_ref, W_ref, AsT_ref, AdT_ref, xw_ref, alsT_ref, aldT_ref):
    x = x_ref[...]
    xw_ref[...] = jnp.dot(x, W_ref[...], preferred_element_type=jnp.float32)
    xt = x.T
    alsT_ref[...] = jnp.dot(AsT_ref[...], xt, preferred_element_type=jnp.float32)
    aldT_ref[...] = jnp.dot(AdT_ref[...], xt, preferred_element_type=jnp.float32)


def _embed1(x_pad, W, AsT, AdT):
    nblk = N_PAD // BLK
    return pl.pallas_call(
        _embed1_body,
        grid=(nblk,),
        in_specs=[
            pl.BlockSpec((BLK, FM), lambda i: (i, 0)),
            pl.BlockSpec((FM, H * FM), lambda i: (0, 0)),
            pl.BlockSpec((H, FM), lambda i: (0, 0)),
            pl.BlockSpec((H, FM), lambda i: (0, 0)),
        ],
        out_specs=(
            pl.BlockSpec((BLK, H * FM), lambda i: (i, 0)),
            pl.BlockSpec((H, BLK), lambda i: (0, i)),
            pl.BlockSpec((H, BLK), lambda i: (0, i)),
        ),
        out_shape=(
            jax.ShapeDtypeStruct((N_PAD, H * FM), jnp.float32),
            jax.ShapeDtypeStruct((H, N_PAD), jnp.float32),
            jax.ShapeDtypeStruct((H, N_PAD), jnp.float32),
        ),
    )(x_pad, W, AsT, AdT)


def _embed2_body(p_ref, b_ref, W_ref, AsT_ref, AdT_ref,
                 x1_ref, xw_ref, alsT_ref, aldT_ref):
    x = jax.nn.relu((p_ref[0] + p_ref[1]) * (1.0 / H) + b_ref[...])
    x1_ref[...] = x
    xw_ref[...] = jnp.dot(x, W_ref[...], preferred_element_type=jnp.float32)
    xt = x.T
    alsT_ref[...] = jnp.dot(AsT_ref[...], xt, preferred_element_type=jnp.float32)
    aldT_ref[...] = jnp.dot(AdT_ref[...], xt, preferred_element_type=jnp.float32)


def _embed2(parts, b, W, AsT, AdT):
    nblk = N_PAD // BLK
    return pl.pallas_call(
        _embed2_body,
        grid=(nblk,),
        in_specs=[
            pl.BlockSpec((2, BLK, FM), lambda i: (0, i, 0)),
            pl.BlockSpec((1, FM), lambda i: (0, 0)),
            pl.BlockSpec((FM, H * FM), lambda i: (0, 0)),
            pl.BlockSpec((H, FM), lambda i: (0, 0)),
            pl.BlockSpec((H, FM), lambda i: (0, 0)),
        ],
        out_specs=(
            pl.BlockSpec((BLK, FM), lambda i: (i, 0)),
            pl.BlockSpec((BLK, H * FM), lambda i: (i, 0)),
            pl.BlockSpec((H, BLK), lambda i: (0, i)),
            pl.BlockSpec((H, BLK), lambda i: (0, i)),
        ),
        out_shape=(
            jax.ShapeDtypeStruct((N_PAD, FM), jnp.float32),
            jax.ShapeDtypeStruct((N_PAD, H * FM), jnp.float32),
            jax.ShapeDtypeStruct((H, N_PAD), jnp.float32),
            jax.ShapeDtypeStruct((H, N_PAD), jnp.float32),
        ),
    )(parts, b, W, AsT, AdT)


def _conv_body(x1_ref, q_ref, b2_ref, Wc1T_ref, Wc2T_ref, cb_ref, xo_ref):
    x2 = jax.nn.relu((q_ref[0] + q_ref[1]) * (1.0 / H) + b2_ref[...])
    xo_ref[...] = (jnp.dot(x1_ref[...], Wc1T_ref[...], preferred_element_type=jnp.float32)
                   + jnp.dot(x2, Wc2T_ref[...], preferred_element_type=jnp.float32)
                   + cb_ref[...])


def _conv(x1, parts2, b2, Wc1T, Wc2T, cb):
    nblk = N_PAD // BLK
    return pl.pallas_call(
        _conv_body,
        grid=(nblk,),
        in_specs=[
            pl.BlockSpec((BLK, FM), lambda i: (i, 0)),
            pl.BlockSpec((2, BLK, FM), lambda i: (0, i, 0)),
            pl.BlockSpec((1, FM), lambda i: (0, 0)),
            pl.BlockSpec((FM, OUT_CH), lambda i: (0, 0)),
            pl.BlockSpec((FM, OUT_CH), lambda i: (0, 0)),
            pl.BlockSpec((1, OUT_CH), lambda i: (0, 0)),
        ],
        out_specs=pl.BlockSpec((BLK, OUT_CH), lambda i: (i, 0)),
        out_shape=jax.ShapeDtypeStruct((N_PAD, OUT_CH), jnp.float32),
    )(x1, parts2, b2, Wc1T, Wc2T, cb)


def _bilinear_body(a_ref, b_ref, o_ref):
    dn = (((1,), (1,)), ((), ()))
    o_ref[...] = lax.dot_general(a_ref[...], b_ref[...], dn,
                                 preferred_element_type=jnp.float32)


def _bilinear(circ_p, mir_p):
    nblk = mir_p.shape[0] // 512
    return pl.pallas_call(
        _bilinear_body,
        grid=(nblk,),
        in_specs=[
            pl.BlockSpec((512, OUT_CH), lambda j: (0, 0)),
            pl.BlockSpec((512, OUT_CH), lambda j: (j, 0)),
        ],
        out_specs=pl.BlockSpec((512, 512), lambda j: (0, j)),
        out_shape=jax.ShapeDtypeStruct((512, nblk * 512), jnp.float32),
    )(circ_p, mir_p)


def _edge_stage(alsT, aldT, xw, src, dst):
    al_s = alsT[:, src].T
    al_d = aldT[:, dst].T
    alpha = jax.nn.leaky_relu(al_s + al_d, negative_slope=NEG_SLOPE)
    ex = jnp.exp(alpha)
    denom = jax.ops.segment_sum(ex, dst, num_segments=N)
    att = ex / denom[dst]
    e2 = src.shape[0]
    contrib = (xw[src].reshape(e2, H, FM) * att[:, :, None]).sum(axis=1)
    out = jax.ops.segment_sum(contrib, dst, num_segments=N)
    out = jnp.pad(out, ((0, N_PAD - N), (0, 0)))
    return jnp.stack([out, jnp.zeros_like(out)])


def kernel(x, edge_index, W1, a_src1, a_dst1, b1, W2, a_src2, a_dst2, b2,
           conv_w, conv_b):
    src = jnp.concatenate([edge_index[0], jnp.arange(N, dtype=edge_index.dtype)])
    dst = jnp.concatenate([edge_index[1], jnp.arange(N, dtype=edge_index.dtype)])

    x_pad = jnp.pad(x, ((0, N_PAD - N), (0, 0)))
    As1T, Ad1T = _prep(W1, a_src1, a_dst1)
    As2T, Ad2T = _prep(W2, a_src2, a_dst2)

    xw1, als1T, ald1T = _embed1(x_pad, W1, As1T, Ad1T)
    parts1 = _edge_stage(als1T, ald1T, xw1, src, dst)

    x1, xw2, als2T, ald2T = _embed2(parts1, b1.reshape(1, FM), W2, As2T, Ad2T)
    parts2 = _edge_stage(als2T, ald2T, xw2, src, dst)

    Wc1T = conv_w[:, 0, :, 0].T
    Wc2T = conv_w[:, 1, :, 0].T
    xo = _conv(x1, parts2, b2.reshape(1, FM), Wc1T, Wc2T,
               conv_b.reshape(1, OUT_CH))

    circ_p = xo[:512]
    mir_p = xo[N_CIRC:N_CIRC + 512 * 19]
    prod = _bilinear(circ_p, mir_p)

    circ = xo[:N_CIRC]
    mirna = xo[N_CIRC:N]
    return prod[:N_CIRC, :N - N_CIRC], circ, mirna
